# baseline (speedup 1.0000x reference)
import functools

import jax
import jax.numpy as jnp
from jax import lax
from jax.experimental import pallas as pl
from jax.experimental.pallas import tpu as pltpu
from jax.experimental.pallas import tpu_sc as plsc

NBR = 5
B, J, D = 250, 200, 128
NC, NS = 2, 16
NW = NC * NS
JC = 40
UNITS = B * (J // JC)

_sc_agg_cache = []


def _get_sc_agg():
    if _sc_agg_cache:
        return _sc_agg_cache[0]
    mesh = plsc.VectorSubcoreMesh(core_axis_name="c", subcore_axis_name="s")

    @functools.partial(
        pl.kernel,
        mesh=mesh,
        out_type=jax.ShapeDtypeStruct((B, J, D), jnp.float32),
        scratch_types=[
            pltpu.VMEM((JC, NBR, D), jnp.float32),
            pltpu.VMEM((JC, D), jnp.float32),
            pltpu.VMEM((8, 16), jnp.float32),
        ],
    )
    def _sc_agg(neigh_hbm, w_hbm, out_hbm, in_v, out_v, w_v):
        wid = lax.axis_index("s") * NC + lax.axis_index("c")
        pltpu.sync_copy(w_hbm, w_v)
        jchunks = J // JC
        base_units = UNITS // NW
        rem = UNITS - base_units * NW
        niter = jnp.where(wid < rem, base_units + 1, base_units)

        def unit_body(i, _):
            u = wid + i * NW
            bi = u // jchunks
            j0 = (u % jchunks) * JC
            pltpu.sync_copy(neigh_hbm.at[bi, pl.ds(j0, JC)], in_v)

            def row_body(rw, _):
                for c in range(D // 16):
                    sl = pl.ds(c * 16, 16)
                    acc = in_v[rw, 0, sl] * w_v[0, :]
                    for k in range(1, NBR):
                        acc = acc + in_v[rw, k, sl] * w_v[k, :]
                    out_v[rw, sl] = acc
                return 0

            lax.fori_loop(0, JC, row_body, 0)
            pltpu.sync_copy(out_v, out_hbm.at[bi, pl.ds(j0, JC)])
            return 0

        lax.fori_loop(0, niter, unit_body, 0)

    _sc_agg_cache.append(_sc_agg)
    return _sc_agg


def _tc_body(x_ref, a_ref, wl_ref, wr_ref, o_ref):
    bb, j, d = x_ref.shape
    r = bb * j
    o_ref[...] = (
        jnp.dot(x_ref[...].reshape(r, d), wl_ref[...],
                preferred_element_type=jnp.float32)
        + jnp.dot(a_ref[...].reshape(r, d), wr_ref[...],
                  preferred_element_type=jnp.float32)
    )


def kernel(x, neigh_x, w_aggr1, W_l, W_r):
    b, j, d = x.shape
    n_rows = b * j
    wsc = jnp.broadcast_to(
        jnp.pad(w_aggr1[0], (0, 8 - NBR))[:, None], (8, 16)
    )
    agg = _get_sc_agg()(neigh_x, wsc)

    bb = 25
    r = bb * j
    out = pl.pallas_call(
        _tc_body,
        grid=(b // bb,),
        in_specs=[
            pl.BlockSpec((bb, j, d), lambda i: (i, 0, 0)),
            pl.BlockSpec((bb, j, d), lambda i: (i, 0, 0)),
            pl.BlockSpec((d, d), lambda i: (0, 0)),
            pl.BlockSpec((d, d), lambda i: (0, 0)),
        ],
        out_specs=pl.BlockSpec((r, d), lambda i: (i, 0)),
        out_shape=jax.ShapeDtypeStruct((n_rows, d), jnp.float32),
        compiler_params=pltpu.CompilerParams(
            dimension_semantics=("arbitrary",),
        ),
    )(x, agg, W_l.T, W_r.T)
    return out


# speedup vs baseline: 1.5896x; 1.5896x over previous
import functools

import jax
import jax.numpy as jnp
from jax import lax
from jax.experimental import pallas as pl
from jax.experimental.pallas import tpu as pltpu
from jax.experimental.pallas import tpu_sc as plsc

NBR = 5
B, J, D = 250, 200, 128
NC, NS = 2, 16
NW = NC * NS
JC = 40
BB = 25
B_TC = 150

_sc_agg_cache = []


def _get_sc_agg():
    if _sc_agg_cache:
        return _sc_agg_cache[0]
    mesh = plsc.VectorSubcoreMesh(core_axis_name="c", subcore_axis_name="s")
    b_sc = B - B_TC
    jchunks = J // JC
    units = b_sc * jchunks

    @functools.partial(
        pl.kernel,
        mesh=mesh,
        out_type=jax.ShapeDtypeStruct((b_sc, J, D), jnp.float32),
        scratch_types=[
            pltpu.VMEM((JC, NBR, D), jnp.float32),
            pltpu.VMEM((JC, D), jnp.float32),
            pltpu.VMEM((8, 16), jnp.float32),
        ],
    )
    def _sc_agg(neigh_hbm, w_hbm, out_hbm, in_v, out_v, w_v):
        wid = lax.axis_index("s") * NC + lax.axis_index("c")
        pltpu.sync_copy(w_hbm, w_v)
        base_units = units // NW
        rem = units - base_units * NW
        niter = jnp.where(wid < rem, base_units + 1, base_units)

        def unit_body(i, _):
            u = wid + i * NW
            bi = u // jchunks
            j0 = (u % jchunks) * JC
            pltpu.sync_copy(neigh_hbm.at[B_TC + bi, pl.ds(j0, JC)], in_v)

            def row_body(rw, _):
                for c in range(D // 16):
                    sl = pl.ds(c * 16, 16)
                    acc = in_v[rw, 0, sl] * w_v[0, :]
                    for k in range(1, NBR):
                        acc = acc + in_v[rw, k, sl] * w_v[k, :]
                    out_v[rw, sl] = acc
                return 0

            lax.fori_loop(0, JC, row_body, 0)
            pltpu.sync_copy(out_v, out_hbm.at[bi, pl.ds(j0, JC)])
            return 0

        lax.fori_loop(0, niter, unit_body, 0)

    _sc_agg_cache.append(_sc_agg)
    return _sc_agg


def _tc_fused_body(x_ref, n_ref, wb_ref, wl_ref, wr_ref, o_ref):
    bb, j, d = x_ref.shape
    r = bb * j
    agg = n_ref[:, :, 0, :] * wb_ref[0, :]
    for k in range(1, NBR):
        agg = agg + n_ref[:, :, k, :] * wb_ref[k, :]
    o_ref[...] = (
        jnp.dot(x_ref[...].reshape(r, d), wl_ref[...],
                preferred_element_type=jnp.float32)
        + jnp.dot(agg.reshape(r, d), wr_ref[...],
                  preferred_element_type=jnp.float32)
    ).reshape(bb, j, d)


def _tc_mm_body(x_ref, a_ref, wl_ref, wr_ref, prev_ref, o_ref):
    del prev_ref
    bb, j, d = x_ref.shape
    r = bb * j
    o_ref[...] = (
        jnp.dot(x_ref[...].reshape(r, d), wl_ref[...],
                preferred_element_type=jnp.float32)
        + jnp.dot(a_ref[...].reshape(r, d), wr_ref[...],
                  preferred_element_type=jnp.float32)
    ).reshape(bb, j, d)


def kernel(x, neigh_x, w_aggr1, W_l, W_r):
    b, j, d = x.shape
    n_rows = b * j
    wl_t = W_l.T
    wr_t = W_r.T
    wsc = jnp.broadcast_to(
        jnp.pad(w_aggr1[0], (0, 8 - NBR))[:, None], (8, 16)
    )
    wb = jnp.pad(
        jnp.broadcast_to(w_aggr1[0][:, None], (NBR, d)), ((0, 8 - NBR), (0, 0))
    )

    agg_sc = _get_sc_agg()(neigh_x, wsc)

    nt = B_TC // BB
    out1 = pl.pallas_call(
        _tc_fused_body,
        grid=(nt,),
        in_specs=[
            pl.BlockSpec((BB, j, d), lambda i: (i, 0, 0)),
            pl.BlockSpec((BB, j, NBR, d), lambda i: (i, 0, 0, 0)),
            pl.BlockSpec((8, d), lambda i: (0, 0)),
            pl.BlockSpec((d, d), lambda i: (0, 0)),
            pl.BlockSpec((d, d), lambda i: (0, 0)),
        ],
        out_specs=pl.BlockSpec((BB, j, d), lambda i: (i, 0, 0)),
        out_shape=jax.ShapeDtypeStruct((b, j, d), jnp.float32),
        compiler_params=pltpu.CompilerParams(
            dimension_semantics=("arbitrary",),
        ),
    )(x, neigh_x, wb, wl_t, wr_t)

    nsc = (b - B_TC) // BB
    out = pl.pallas_call(
        _tc_mm_body,
        grid=(nsc,),
        in_specs=[
            pl.BlockSpec((BB, j, d), lambda i: (i + B_TC // BB, 0, 0)),
            pl.BlockSpec((BB, j, d), lambda i: (i, 0, 0)),
            pl.BlockSpec((d, d), lambda i: (0, 0)),
            pl.BlockSpec((d, d), lambda i: (0, 0)),
            pl.BlockSpec(memory_space=pl.ANY),
        ],
        out_specs=pl.BlockSpec((BB, j, d), lambda i: (i + B_TC // BB, 0, 0)),
        out_shape=jax.ShapeDtypeStruct((b, j, d), jnp.float32),
        input_output_aliases={4: 0},
        compiler_params=pltpu.CompilerParams(
            dimension_semantics=("arbitrary",),
        ),
    )(x, agg_sc, wl_t, wr_t, out1)
    return out.reshape(n_rows, d)


# speedup vs baseline: 1.8379x; 1.1562x over previous
import jax
import jax.numpy as jnp
from jax.experimental import pallas as pl
from jax.experimental.pallas import tpu as pltpu

NBR = 5
B_PER_BLOCK = 25


def _body(x_ref, n_ref, wb_ref, wl_ref, wr_ref, o_ref):
    bb, j, d = x_ref.shape
    r = bb * j
    agg = n_ref[:, :, 0, :] * wb_ref[0, :]
    for k in range(1, NBR):
        agg = agg + n_ref[:, :, k, :] * wb_ref[k, :]
    xb = x_ref[...].reshape(r, d)
    aggb = agg.reshape(r, d)
    o_ref[...] = (
        jnp.dot(xb, wl_ref[...], preferred_element_type=jnp.float32)
        + jnp.dot(aggb, wr_ref[...], preferred_element_type=jnp.float32)
    ).reshape(bb, j, d)


def kernel(x, neigh_x, w_aggr1, W_l, W_r):
    b, j, d = x.shape
    n_rows = b * j
    wb = jnp.pad(
        jnp.broadcast_to(w_aggr1[0][:, None], (NBR, d)), ((0, 8 - NBR), (0, 0))
    )
    wl_t = W_l.T
    wr_t = W_r.T

    bb = 50
    jb = 40
    grid = (b // bb, j // jb)
    out = pl.pallas_call(
        _body,
        grid=grid,
        in_specs=[
            pl.BlockSpec((bb, jb, d), lambda i, s: (i, s, 0)),
            pl.BlockSpec((bb, jb, NBR, d), lambda i, s: (i, s, 0, 0)),
            pl.BlockSpec((8, d), lambda i, s: (0, 0)),
            pl.BlockSpec((d, d), lambda i, s: (0, 0)),
            pl.BlockSpec((d, d), lambda i, s: (0, 0)),
        ],
        out_specs=pl.BlockSpec((bb, jb, d), lambda i, s: (i, s, 0)),
        out_shape=jax.ShapeDtypeStruct((b, j, d), jnp.float32),
        compiler_params=pltpu.CompilerParams(
            dimension_semantics=("arbitrary", "arbitrary"),
        ),
    )(x, neigh_x, wb, wl_t, wr_t)
    return out.reshape(n_rows, d)


# speedup vs baseline: 1.9046x; 1.0363x over previous
import jax
import jax.numpy as jnp
from jax.experimental import pallas as pl
from jax.experimental.pallas import tpu as pltpu

NBR = 5
B_PER_BLOCK = 25


def _body(x_ref, n_ref, wb_ref, wl_ref, wr_ref, o_ref):
    bb, j, d = x_ref.shape
    r = bb * j
    agg = n_ref[:, :, 0, :] * wb_ref[0, :]
    for k in range(1, NBR):
        agg = agg + n_ref[:, :, k, :] * wb_ref[k, :]
    xb = x_ref[...].reshape(r, d)
    aggb = agg.reshape(r, d)
    o_ref[...] = (
        jnp.dot(xb, wl_ref[...], preferred_element_type=jnp.float32)
        + jnp.dot(aggb, wr_ref[...], preferred_element_type=jnp.float32)
    )


def kernel(x, neigh_x, w_aggr1, W_l, W_r):
    b, j, d = x.shape
    n_rows = b * j
    wb = jnp.pad(
        jnp.broadcast_to(w_aggr1[0][:, None], (NBR, d)), ((0, 8 - NBR), (0, 0))
    )
    wl_t = W_l.T
    wr_t = W_r.T

    bb = B_PER_BLOCK
    r = bb * j
    grid = (b // bb,)
    out = pl.pallas_call(
        _body,
        grid=grid,
        in_specs=[
            pl.BlockSpec((bb, j, d), lambda i: (i, 0, 0)),
            pl.BlockSpec((bb, j, NBR, d), lambda i: (i, 0, 0, 0)),
            pl.BlockSpec((8, d), lambda i: (0, 0)),
            pl.BlockSpec((d, d), lambda i: (0, 0)),
            pl.BlockSpec((d, d), lambda i: (0, 0)),
        ],
        out_specs=pl.BlockSpec((r, d), lambda i: (i, 0)),
        out_shape=jax.ShapeDtypeStruct((n_rows, d), jnp.float32),
        compiler_params=pltpu.CompilerParams(
            dimension_semantics=("arbitrary",),
        ),
    )(x, neigh_x, wb, wl_t, wr_t)
    return out
